# initial kernel scaffold (unmeasured)
import jax
import jax.numpy as jnp
from jax import lax
from jax.experimental import pallas as pl
from jax.experimental.pallas import tpu as pltpu

N_DEV = 32
M, NCOLS = 4096, 8192
CHUNK = M // N_DEV


def _logical_order():
    order = []
    for z in range(4):
        for yi in range(4):
            xs = (0, 1) if yi % 2 == 0 else (1, 0)
            for x in xs:
                order.append((x, yi, z))
    return order


def _ring_order():
    half = []
    for zi in range(4):
        ys = range(4) if zi % 2 == 0 else range(3, -1, -1)
        for y in ys:
            half.append((0, y, zi))
    return half + [(1, y, z) for (_, y, z) in reversed(half)]


_LOG = _logical_order()
_RING = _ring_order()
_L_OF_COORD = {c: l for l, c in enumerate(_LOG)}
_RING_L = [_L_OF_COORD[c] for c in _RING]
_RPOS = [0] * N_DEV
for p, l in enumerate(_RING_L):
    _RPOS[l] = p
_NEXT = [0] * N_DEV
_PREV = [0] * N_DEV
for p, l in enumerate(_RING_L):
    _NEXT[l] = _RING_L[(p + 1) % N_DEV]
    _PREV[l] = _RING_L[(p - 1) % N_DEV]


def _body(part_ref, r_ref, nxt_ref, prv_ref, out_ref, amax_ref,
          acc, recv, lc, send_sems, recv_sems, lc_sem, out_sem,
          credit_sem):
    r = r_ref[0]
    nxt = nxt_ref[0]
    prv = prv_ref[0]

    def _rdma(src, dst_slot, slot):
        return pltpu.make_async_remote_copy(
            src_ref=src,
            dst_ref=recv.at[dst_slot],
            send_sem=send_sems.at[slot],
            recv_sem=recv_sems.at[slot],
            device_id=(nxt,),
            device_id_type=pl.DeviceIdType.MESH,
        )

    def _credit_to_prev():
        pl.semaphore_signal(credit_sem, inc=1, device_id=(prv,),
                            device_id_type=pl.DeviceIdType.MESH)

    c0 = (r - 1) % N_DEV
    pre = pltpu.make_async_copy(
        part_ref.at[pl.ds(c0 * CHUNK, CHUNK), :], acc.at[0], lc_sem)
    pre.start()

    barrier_sem = pltpu.get_barrier_semaphore()
    pl.semaphore_signal(barrier_sem, inc=1, device_id=(nxt,),
                        device_id_type=pl.DeviceIdType.MESH)
    pl.semaphore_signal(barrier_sem, inc=1, device_id=(prv,),
                        device_id_type=pl.DeviceIdType.MESH)
    pl.semaphore_wait(barrier_sem, 2)
    pre.wait()

    for s in range(N_DEV - 1):
        slot = s % 2
        if s >= 2:
            pl.semaphore_wait(credit_sem, 1)
        rdma = _rdma(acc.at[slot], slot, slot)
        rdma.start()
        c = (r - 2 - s) % N_DEV
        ldma = pltpu.make_async_copy(
            part_ref.at[pl.ds(c * CHUNK, CHUNK), :], lc, lc_sem)
        ldma.start()
        rdma.wait()
        ldma.wait()
        if s < N_DEV - 2:
            acc[1 - slot] = recv[slot] + lc[...]
        else:
            acc[1 - slot] = jnp.maximum(recv[slot] + lc[...], 0.0)
        _credit_to_prev()

    amax_val = jnp.max(acc[1])
    own = pltpu.make_async_copy(
        acc.at[1], out_ref.at[pl.ds(r * CHUNK, CHUNK), :], out_sem)
    own.start()
    own.wait()

    for t in range(N_DEV - 1):
        g = (N_DEV - 1) + t
        slot = g % 2
        pl.semaphore_wait(credit_sem, 1)
        src = acc.at[1] if t == 0 else recv.at[1 - slot]
        rdma = _rdma(src, slot, slot)
        rdma.start()
        rdma.wait()
        if t > 0:
            _credit_to_prev()
        origin = (r - 1 - t) % N_DEV
        st = pltpu.make_async_copy(
            recv.at[slot], out_ref.at[pl.ds(origin * CHUNK, CHUNK), :],
            out_sem)
        st.start()
        amax_val = jnp.maximum(amax_val, jnp.max(recv[slot]))
        st.wait()
    _credit_to_prev()

    amax_ref[0, 0] = amax_val
    pl.semaphore_wait(credit_sem, 2)


def _all_reduce_relu_amax(partial, r, nxt, prv):
    return pl.pallas_call(
        _body,
        out_shape=[
            jax.ShapeDtypeStruct((M, NCOLS), jnp.float32),
            jax.ShapeDtypeStruct((1, 1), jnp.float32),
        ],
        in_specs=[
            pl.BlockSpec(memory_space=pl.ANY),
            pl.BlockSpec(memory_space=pltpu.SMEM),
            pl.BlockSpec(memory_space=pltpu.SMEM),
            pl.BlockSpec(memory_space=pltpu.SMEM),
        ],
        out_specs=[
            pl.BlockSpec(memory_space=pl.ANY),
            pl.BlockSpec(memory_space=pltpu.SMEM),
        ],
        scratch_shapes=[
            pltpu.VMEM((2, CHUNK, NCOLS), jnp.float32),
            pltpu.VMEM((2, CHUNK, NCOLS), jnp.float32),
            pltpu.VMEM((CHUNK, NCOLS), jnp.float32),
            pltpu.SemaphoreType.DMA((2,)),
            pltpu.SemaphoreType.DMA((2,)),
            pltpu.SemaphoreType.DMA,
            pltpu.SemaphoreType.DMA,
            pltpu.SemaphoreType.REGULAR,
        ],
        compiler_params=pltpu.CompilerParams(collective_id=0),
    )(partial, r, nxt, prv)


def kernel(x, w_mat):
    partial = lax.dot_general(
        x, w_mat, (((1,), (0,)), ((), ())),
        precision=lax.Precision.HIGHEST,
        preferred_element_type=jnp.float32,
    )
    i = lax.axis_index("i")
    r = jnp.asarray(_RPOS, jnp.int32)[i].reshape(1)
    nxt = jnp.asarray(_NEXT, jnp.int32)[i].reshape(1)
    prv = jnp.asarray(_PREV, jnp.int32)[i].reshape(1)
    y, amax = _all_reduce_relu_amax(partial, r, nxt, prv)

    scale = amax[0, 0] / 448.0
    q = jnp.minimum(y / scale, 448.0)
    q = q.astype(jnp.float8_e4m3fn).astype(jnp.float32)
    return q * scale


# baseline (device time: 3225772 ns/iter reference)
import jax
import jax.numpy as jnp
from jax import lax
from jax.experimental import pallas as pl
from jax.experimental.pallas import tpu as pltpu

N_DEV = 32
M, NCOLS = 4096, 8192
CHUNK = M // N_DEV


def _logical_order():
    order = []
    for z in range(4):
        for yi in range(4):
            xs = (0, 1) if yi % 2 == 0 else (1, 0)
            for x in xs:
                order.append((x, yi, z))
    return order


def _ring_order():
    half = []
    for zi in range(4):
        ys = range(4) if zi % 2 == 0 else range(3, -1, -1)
        for y in ys:
            half.append((0, y, zi))
    return half + [(1, y, z) for (_, y, z) in reversed(half)]


_LOG = _logical_order()
_RING = _ring_order()
_L_OF_COORD = {c: l for l, c in enumerate(_LOG)}
_RING_L = [_L_OF_COORD[c] for c in _RING]
_RPOS = [0] * N_DEV
for p, l in enumerate(_RING_L):
    _RPOS[l] = p
_NEXT = [0] * N_DEV
_PREV = [0] * N_DEV
for p, l in enumerate(_RING_L):
    _NEXT[l] = _RING_L[(p + 1) % N_DEV]
    _PREV[l] = _RING_L[(p - 1) % N_DEV]


def _body(part_ref, r_ref, nxt_ref, prv_ref, out_ref, amax_ref,
          acc, recv, lc, send_sems, recv_sems, lc_sem, out_sem,
          credit_sem):
    r = r_ref[0]
    nxt = nxt_ref[0]
    prv = prv_ref[0]

    def _rdma(src, dst_slot, slot):
        return pltpu.make_async_remote_copy(
            src_ref=src,
            dst_ref=recv.at[dst_slot],
            send_sem=send_sems.at[slot],
            recv_sem=recv_sems.at[slot],
            device_id=(nxt,),
            device_id_type=pl.DeviceIdType.MESH,
        )

    def _credit_to_prev():
        pl.semaphore_signal(credit_sem, inc=1, device_id=(prv,),
                            device_id_type=pl.DeviceIdType.MESH)

    c0 = (r - 1) % N_DEV
    pre = pltpu.make_async_copy(
        part_ref.at[pl.ds(c0 * CHUNK, CHUNK), :], acc.at[0], lc_sem)
    pre.start()

    barrier_sem = pltpu.get_barrier_semaphore()
    pl.semaphore_signal(barrier_sem, inc=1, device_id=(nxt,),
                        device_id_type=pl.DeviceIdType.MESH)
    pl.semaphore_signal(barrier_sem, inc=1, device_id=(prv,),
                        device_id_type=pl.DeviceIdType.MESH)
    pl.semaphore_wait(barrier_sem, 2)
    pre.wait()

    for s in range(N_DEV - 1):
        slot = s % 2
        if s >= 2:
            pl.semaphore_wait(credit_sem, 1)
        rdma = _rdma(acc.at[slot], slot, slot)
        rdma.start()
        c = (r - 2 - s) % N_DEV
        ldma = pltpu.make_async_copy(
            part_ref.at[pl.ds(c * CHUNK, CHUNK), :], lc, lc_sem)
        ldma.start()
        rdma.wait()
        ldma.wait()
        if s < N_DEV - 2:
            acc[1 - slot] = recv[slot] + lc[...]
        else:
            acc[1 - slot] = jnp.maximum(recv[slot] + lc[...], 0.0)
        _credit_to_prev()

    amax_val = jnp.max(acc[1])
    own = pltpu.make_async_copy(
        acc.at[1], out_ref.at[pl.ds(r * CHUNK, CHUNK), :], out_sem)
    own.start()
    own.wait()

    for t in range(N_DEV - 1):
        g = (N_DEV - 1) + t
        slot = g % 2
        pl.semaphore_wait(credit_sem, 1)
        src = acc.at[1] if t == 0 else recv.at[1 - slot]
        rdma = _rdma(src, slot, slot)
        rdma.start()
        rdma.wait()
        if t > 0:
            _credit_to_prev()
        origin = (r - 1 - t) % N_DEV
        st = pltpu.make_async_copy(
            recv.at[slot], out_ref.at[pl.ds(origin * CHUNK, CHUNK), :],
            out_sem)
        st.start()
        amax_val = jnp.maximum(amax_val, jnp.max(recv[slot]))
        st.wait()
    _credit_to_prev()

    amax_ref[0, 0] = amax_val
    pl.semaphore_wait(credit_sem, 2)


def _all_reduce_relu_amax(partial, r, nxt, prv):
    return pl.pallas_call(
        _body,
        out_shape=[
            jax.ShapeDtypeStruct((M, NCOLS), jnp.float32),
            jax.ShapeDtypeStruct((1, 1), jnp.float32),
        ],
        in_specs=[
            pl.BlockSpec(memory_space=pl.ANY),
            pl.BlockSpec(memory_space=pltpu.SMEM),
            pl.BlockSpec(memory_space=pltpu.SMEM),
            pl.BlockSpec(memory_space=pltpu.SMEM),
        ],
        out_specs=[
            pl.BlockSpec(memory_space=pl.ANY),
            pl.BlockSpec(memory_space=pltpu.SMEM),
        ],
        scratch_shapes=[
            pltpu.VMEM((2, CHUNK, NCOLS), jnp.float32),
            pltpu.VMEM((2, CHUNK, NCOLS), jnp.float32),
            pltpu.VMEM((CHUNK, NCOLS), jnp.float32),
            pltpu.SemaphoreType.DMA((2,)),
            pltpu.SemaphoreType.DMA((2,)),
            pltpu.SemaphoreType.DMA,
            pltpu.SemaphoreType.DMA,
            pltpu.SemaphoreType.REGULAR,
        ],
        compiler_params=pltpu.CompilerParams(collective_id=0),
    )(partial, r, nxt, prv)


def kernel(x, w_mat):
    partial = lax.dot_general(
        x, w_mat, (((1,), (0,)), ((), ())),
        precision=lax.Precision.HIGHEST,
        preferred_element_type=jnp.float32,
    )
    i = lax.axis_index("i")
    r = jnp.asarray(_RPOS, jnp.int32)[i].reshape(1)
    nxt = jnp.asarray(_NEXT, jnp.int32)[i].reshape(1)
    prv = jnp.asarray(_PREV, jnp.int32)[i].reshape(1)
    y, amax = _all_reduce_relu_amax(partial, r, nxt, prv)

    scale = amax[0, 0] / 448.0
    v = jnp.minimum(y / scale, 448.0)
    bits = lax.bitcast_convert_type(v, jnp.int32)
    rb = (bits + 0x7FFFF + ((bits >> 20) & 1)) & ~0xFFFFF
    norm = lax.bitcast_convert_type(rb, jnp.float32)
    sub = jnp.round(v * 512.0) * jnp.float32(1.0 / 512.0)
    q = jnp.where(v < 2.0 ** -6, sub, norm)
    return q * scale


# device time: 1855925 ns/iter; 1.7381x vs baseline; 1.7381x over previous
import jax
import jax.numpy as jnp
from jax import lax
from jax.experimental import pallas as pl
from jax.experimental.pallas import tpu as pltpu

N_DEV = 32
M, NCOLS = 4096, 8192
HALF = NCOLS // 2
CHUNK = M // N_DEV


def _logical_order():
    order = []
    for z in range(4):
        for yi in range(4):
            xs = (0, 1) if yi % 2 == 0 else (1, 0)
            for x in xs:
                order.append((x, yi, z))
    return order


def _ring_order():
    half = []
    for zi in range(4):
        ys = range(4) if zi % 2 == 0 else range(3, -1, -1)
        for y in ys:
            half.append((0, y, zi))
    return half + [(1, y, z) for (_, y, z) in reversed(half)]


_LOG = _logical_order()
_RING = _ring_order()
_L_OF_COORD = {c: l for l, c in enumerate(_LOG)}
_RING_L = [_L_OF_COORD[c] for c in _RING]
_RPOS = [0] * N_DEV
for p, l in enumerate(_RING_L):
    _RPOS[l] = p
_NEXT = [0] * N_DEV
_PREV = [0] * N_DEV
for p, l in enumerate(_RING_L):
    _NEXT[l] = _RING_L[(p + 1) % N_DEV]
    _PREV[l] = _RING_L[(p - 1) % N_DEV]


class _Dir:

    def __init__(self, r, to, fro, col_off, acc, recv, lc,
                 send_sems, recv_sems, lc_sem, out_sem, credit_sem):
        self.r = r
        self.to = to
        self.fro = fro
        self.col_off = col_off
        self.acc = acc
        self.recv = recv
        self.lc = lc
        self.send_sems = send_sems
        self.recv_sems = recv_sems
        self.lc_sem = lc_sem
        self.out_sem = out_sem
        self.credit_sem = credit_sem

    def rdma(self, src, slot):
        return pltpu.make_async_remote_copy(
            src_ref=src,
            dst_ref=self.recv.at[slot],
            send_sem=self.send_sems.at[slot],
            recv_sem=self.recv_sems.at[slot],
            device_id=(self.to,),
            device_id_type=pl.DeviceIdType.MESH,
        )

    def credit(self):
        pl.semaphore_signal(self.credit_sem, inc=1, device_id=(self.fro,),
                            device_id_type=pl.DeviceIdType.MESH)


def _body(part_ref, r_ref, nxt_ref, prv_ref, out_ref, amax_ref,
          acc_f, recv_f, lc_f, acc_b, recv_b, lc_b,
          send_f, recv_sf, lc_sf, out_sf, cred_f,
          send_b, recv_sb, lc_sb, out_sb, cred_b):
    r = r_ref[0]
    nxt = nxt_ref[0]
    prv = prv_ref[0]
    r_b = (N_DEV - r) % N_DEV

    F = _Dir(r, nxt, prv, 0, acc_f, recv_f, lc_f,
             send_f, recv_sf, lc_sf, out_sf, cred_f)
    B = _Dir(r_b, prv, nxt, HALF, acc_b, recv_b, lc_b,
             send_b, recv_sb, lc_sb, out_sb, cred_b)
    dirs = (F, B)

    pres = []
    for d in dirs:
        c0 = (d.r - 1) % N_DEV
        pre = pltpu.make_async_copy(
            part_ref.at[pl.ds(c0 * CHUNK, CHUNK), pl.ds(d.col_off, HALF)],
            d.acc.at[0], d.lc_sem)
        pre.start()
        pres.append(pre)

    barrier_sem = pltpu.get_barrier_semaphore()
    pl.semaphore_signal(barrier_sem, inc=1, device_id=(nxt,),
                        device_id_type=pl.DeviceIdType.MESH)
    pl.semaphore_signal(barrier_sem, inc=1, device_id=(prv,),
                        device_id_type=pl.DeviceIdType.MESH)
    pl.semaphore_wait(barrier_sem, 2)
    for pre in pres:
        pre.wait()

    for s in range(N_DEV - 1):
        slot = s % 2
        if s >= 2:
            for d in dirs:
                pl.semaphore_wait(d.credit_sem, 1)
        rdmas = []
        for d in dirs:
            rd = d.rdma(d.acc.at[slot], slot)
            rd.start()
            rdmas.append(rd)
        ldmas = []
        for d in dirs:
            c = (d.r - 2 - s) % N_DEV
            ld = pltpu.make_async_copy(
                part_ref.at[pl.ds(c * CHUNK, CHUNK), pl.ds(d.col_off, HALF)],
                d.lc, d.lc_sem)
            ld.start()
            ldmas.append(ld)
        for d, rd, ld in zip(dirs, rdmas, ldmas):
            rd.wait()
            ld.wait()
            if s < N_DEV - 2:
                d.acc[1 - slot] = d.recv[slot] + d.lc[...]
            else:
                d.acc[1 - slot] = jnp.maximum(d.recv[slot] + d.lc[...], 0.0)
            d.credit()

    amax_val = jnp.maximum(jnp.max(F.acc[1]), jnp.max(B.acc[1]))
    for d in dirs:
        own = pltpu.make_async_copy(
            d.acc.at[1],
            out_ref.at[pl.ds(d.r * CHUNK, CHUNK), pl.ds(d.col_off, HALF)],
            d.out_sem)
        own.start()
        own.wait()

    for t in range(N_DEV - 1):
        slot = (N_DEV - 1 + t) % 2
        for d in dirs:
            pl.semaphore_wait(d.credit_sem, 1)
        rdmas = []
        for d in dirs:
            src = d.acc.at[1] if t == 0 else d.recv.at[1 - slot]
            rd = d.rdma(src, slot)
            rd.start()
            rdmas.append(rd)
        for d, rd in zip(dirs, rdmas):
            rd.wait()
            if t > 0:
                d.credit()
        for d in dirs:
            origin = (d.r - 1 - t) % N_DEV
            st = pltpu.make_async_copy(
                d.recv.at[slot],
                out_ref.at[pl.ds(origin * CHUNK, CHUNK),
                           pl.ds(d.col_off, HALF)],
                d.out_sem)
            st.start()
            amax_val = jnp.maximum(amax_val, jnp.max(d.recv[slot]))
            st.wait()
    for d in dirs:
        d.credit()

    amax_ref[0, 0] = amax_val
    for d in dirs:
        pl.semaphore_wait(d.credit_sem, 2)


def _all_reduce_relu_amax(partial, r, nxt, prv):
    dir_scratch = [
        pltpu.VMEM((2, CHUNK, HALF), jnp.float32),
        pltpu.VMEM((2, CHUNK, HALF), jnp.float32),
        pltpu.VMEM((CHUNK, HALF), jnp.float32),
    ]
    dir_sems = [
        pltpu.SemaphoreType.DMA((2,)),
        pltpu.SemaphoreType.DMA((2,)),
        pltpu.SemaphoreType.DMA,
        pltpu.SemaphoreType.DMA,
        pltpu.SemaphoreType.REGULAR,
    ]
    return pl.pallas_call(
        _body,
        out_shape=[
            jax.ShapeDtypeStruct((M, NCOLS), jnp.float32),
            jax.ShapeDtypeStruct((1, 1), jnp.float32),
        ],
        in_specs=[
            pl.BlockSpec(memory_space=pl.ANY),
            pl.BlockSpec(memory_space=pltpu.SMEM),
            pl.BlockSpec(memory_space=pltpu.SMEM),
            pl.BlockSpec(memory_space=pltpu.SMEM),
        ],
        out_specs=[
            pl.BlockSpec(memory_space=pl.ANY),
            pl.BlockSpec(memory_space=pltpu.SMEM),
        ],
        scratch_shapes=(dir_scratch + dir_scratch
                        + dir_sems + dir_sems),
        compiler_params=pltpu.CompilerParams(collective_id=0),
    )(partial, r, nxt, prv)


def kernel(x, w_mat):
    partial = lax.dot_general(
        x, w_mat, (((1,), (0,)), ((), ())),
        precision=lax.Precision.HIGHEST,
        preferred_element_type=jnp.float32,
    )
    i = lax.axis_index("i")
    r = jnp.asarray(_RPOS, jnp.int32)[i].reshape(1)
    nxt = jnp.asarray(_NEXT, jnp.int32)[i].reshape(1)
    prv = jnp.asarray(_PREV, jnp.int32)[i].reshape(1)
    y, amax = _all_reduce_relu_amax(partial, r, nxt, prv)

    scale = amax[0, 0] / 448.0
    v = jnp.minimum(y / scale, 448.0)
    bits = lax.bitcast_convert_type(v, jnp.int32)
    rb = (bits + 0x7FFFF + ((bits >> 20) & 1)) & ~0xFFFFF
    norm = lax.bitcast_convert_type(rb, jnp.float32)
    sub = jnp.round(v * 512.0) * jnp.float32(1.0 / 512.0)
    q = jnp.where(v < 2.0 ** -6, sub, norm)
    return q * scale


# device time: 1761698 ns/iter; 1.8311x vs baseline; 1.0535x over previous
import jax
import jax.numpy as jnp
from jax import lax
from jax.experimental import pallas as pl
from jax.experimental.pallas import tpu as pltpu

N_DEV = 32
M, NCOLS = 4096, 8192
HALF = NCOLS // 2
CHUNK = M // N_DEV


def _logical_order():
    order = []
    for z in range(4):
        for yi in range(4):
            xs = (0, 1) if yi % 2 == 0 else (1, 0)
            for x in xs:
                order.append((x, yi, z))
    return order


def _ring_order():
    half = []
    for zi in range(4):
        ys = range(4) if zi % 2 == 0 else range(3, -1, -1)
        for y in ys:
            half.append((0, y, zi))
    return half + [(1, y, z) for (_, y, z) in reversed(half)]


_LOG = _logical_order()
_RING = _ring_order()
_L_OF_COORD = {c: l for l, c in enumerate(_LOG)}
_RING_L = [_L_OF_COORD[c] for c in _RING]
_RPOS = [0] * N_DEV
for p, l in enumerate(_RING_L):
    _RPOS[l] = p
_NEXT = [0] * N_DEV
_PREV = [0] * N_DEV
for p, l in enumerate(_RING_L):
    _NEXT[l] = _RING_L[(p + 1) % N_DEV]
    _PREV[l] = _RING_L[(p - 1) % N_DEV]


class _Dir:

    def __init__(self, r, to, fro, col_off, acc, recv, lc,
                 send_sems, recv_sems, lc_sem, out_sem, credit_sem):
        self.r = r
        self.to = to
        self.fro = fro
        self.col_off = col_off
        self.acc = acc
        self.recv = recv
        self.lc = lc
        self.send_sems = send_sems
        self.recv_sems = recv_sems
        self.lc_sem = lc_sem
        self.out_sem = out_sem
        self.credit_sem = credit_sem

    def rdma(self, src, slot):
        return pltpu.make_async_remote_copy(
            src_ref=src,
            dst_ref=self.recv.at[slot],
            send_sem=self.send_sems.at[slot],
            recv_sem=self.recv_sems.at[slot],
            device_id=(self.to,),
            device_id_type=pl.DeviceIdType.MESH,
        )

    def credit(self):
        pl.semaphore_signal(self.credit_sem, inc=1, device_id=(self.fro,),
                            device_id_type=pl.DeviceIdType.MESH)


def _body(part_ref, r_ref, nxt_ref, prv_ref, out_ref, amax_ref,
          acc_f, recv_f, lc_f, acc_b, recv_b, lc_b,
          send_f, recv_sf, lc_sf, out_sf, cred_f,
          send_b, recv_sb, lc_sb, out_sb, cred_b):
    r = r_ref[0]
    nxt = nxt_ref[0]
    prv = prv_ref[0]
    r_b = (N_DEV - r) % N_DEV

    F = _Dir(r, nxt, prv, 0, acc_f, recv_f, lc_f,
             send_f, recv_sf, lc_sf, out_sf, cred_f)
    B = _Dir(r_b, prv, nxt, HALF, acc_b, recv_b, lc_b,
             send_b, recv_sb, lc_sb, out_sb, cred_b)
    dirs = (F, B)

    pres = []
    for d in dirs:
        c0 = (d.r - 1) % N_DEV
        pre = pltpu.make_async_copy(
            part_ref.at[pl.ds(c0 * CHUNK, CHUNK), pl.ds(d.col_off, HALF)],
            d.acc.at[0], d.lc_sem)
        pre.start()
        pres.append(pre)

    barrier_sem = pltpu.get_barrier_semaphore()
    pl.semaphore_signal(barrier_sem, inc=1, device_id=(nxt,),
                        device_id_type=pl.DeviceIdType.MESH)
    pl.semaphore_signal(barrier_sem, inc=1, device_id=(prv,),
                        device_id_type=pl.DeviceIdType.MESH)
    pl.semaphore_wait(barrier_sem, 2)
    for pre in pres:
        pre.wait()

    for s in range(N_DEV - 1):
        slot = s % 2
        if s >= 2:
            for d in dirs:
                pl.semaphore_wait(d.credit_sem, 1)
        rdmas = []
        for d in dirs:
            rd = d.rdma(d.acc.at[slot], slot)
            rd.start()
            rdmas.append(rd)
        ldmas = []
        for d in dirs:
            c = (d.r - 2 - s) % N_DEV
            ld = pltpu.make_async_copy(
                part_ref.at[pl.ds(c * CHUNK, CHUNK), pl.ds(d.col_off, HALF)],
                d.lc, d.lc_sem)
            ld.start()
            ldmas.append(ld)
        for d, rd, ld in zip(dirs, rdmas, ldmas):
            rd.wait()
            ld.wait()
            if s < N_DEV - 2:
                d.acc[1 - slot] = d.recv[slot] + d.lc[...]
            else:
                d.acc[1 - slot] = jnp.maximum(d.recv[slot] + d.lc[...], 0.0)
            d.credit()

    amax_val = jnp.maximum(jnp.max(F.acc[1]), jnp.max(B.acc[1]))

    for t in range(N_DEV - 1):
        slot = (N_DEV - 1 + t) % 2
        for d in dirs:
            pl.semaphore_wait(d.credit_sem, 1)
        rdmas = []
        for d in dirs:
            src = d.acc.at[1] if t == 0 else d.recv.at[1 - slot]
            rd = d.rdma(src, slot)
            rd.start()
            rdmas.append(rd)
        sts = []
        for d in dirs:
            if t == 0:
                src, row = d.acc.at[1], d.r * CHUNK
            else:
                origin = (d.r - t) % N_DEV
                src, row = d.recv.at[1 - slot], origin * CHUNK
                amax_val = jnp.maximum(amax_val, jnp.max(d.recv[1 - slot]))
            st = pltpu.make_async_copy(
                src, out_ref.at[pl.ds(row, CHUNK), pl.ds(d.col_off, HALF)],
                d.out_sem)
            st.start()
            sts.append(st)
        for st in sts:
            st.wait()
        for d, rd in zip(dirs, rdmas):
            rd.wait()
            if t > 0:
                d.credit()
    slot = (2 * N_DEV - 3) % 2
    sts = []
    for d in dirs:
        origin = (d.r - (N_DEV - 1)) % N_DEV
        st = pltpu.make_async_copy(
            d.recv.at[slot],
            out_ref.at[pl.ds(origin * CHUNK, CHUNK), pl.ds(d.col_off, HALF)],
            d.out_sem)
        st.start()
        amax_val = jnp.maximum(amax_val, jnp.max(d.recv[slot]))
        sts.append(st)
    for st, d in zip(sts, dirs):
        st.wait()
        d.credit()

    amax_ref[0, 0] = amax_val
    for d in dirs:
        pl.semaphore_wait(d.credit_sem, 2)


def _all_reduce_relu_amax(partial, r, nxt, prv):
    dir_scratch = [
        pltpu.VMEM((2, CHUNK, HALF), jnp.float32),
        pltpu.VMEM((2, CHUNK, HALF), jnp.float32),
        pltpu.VMEM((CHUNK, HALF), jnp.float32),
    ]
    dir_sems = [
        pltpu.SemaphoreType.DMA((2,)),
        pltpu.SemaphoreType.DMA((2,)),
        pltpu.SemaphoreType.DMA,
        pltpu.SemaphoreType.DMA,
        pltpu.SemaphoreType.REGULAR,
    ]
    return pl.pallas_call(
        _body,
        out_shape=[
            jax.ShapeDtypeStruct((M, NCOLS), jnp.float32),
            jax.ShapeDtypeStruct((1, 1), jnp.float32),
        ],
        in_specs=[
            pl.BlockSpec(memory_space=pl.ANY),
            pl.BlockSpec(memory_space=pltpu.SMEM),
            pl.BlockSpec(memory_space=pltpu.SMEM),
            pl.BlockSpec(memory_space=pltpu.SMEM),
        ],
        out_specs=[
            pl.BlockSpec(memory_space=pl.ANY),
            pl.BlockSpec(memory_space=pltpu.SMEM),
        ],
        scratch_shapes=(dir_scratch + dir_scratch
                        + dir_sems + dir_sems),
        compiler_params=pltpu.CompilerParams(collective_id=0),
    )(partial, r, nxt, prv)


def kernel(x, w_mat):
    partial = lax.dot_general(
        x, w_mat, (((1,), (0,)), ((), ())),
        precision=lax.Precision.HIGHEST,
        preferred_element_type=jnp.float32,
    )
    i = lax.axis_index("i")
    r = jnp.asarray(_RPOS, jnp.int32)[i].reshape(1)
    nxt = jnp.asarray(_NEXT, jnp.int32)[i].reshape(1)
    prv = jnp.asarray(_PREV, jnp.int32)[i].reshape(1)
    y, amax = _all_reduce_relu_amax(partial, r, nxt, prv)

    scale = amax[0, 0] / 448.0
    v = jnp.minimum(y / scale, 448.0)
    bits = lax.bitcast_convert_type(v, jnp.int32)
    rb = (bits + 0x7FFFF + ((bits >> 20) & 1)) & ~0xFFFFF
    norm = lax.bitcast_convert_type(rb, jnp.float32)
    sub = jnp.round(v * 512.0) * jnp.float32(1.0 / 512.0)
    q = jnp.where(v < 2.0 ** -6, sub, norm)
    return q * scale


# device time: 1701464 ns/iter; 1.8959x vs baseline; 1.0354x over previous
import jax
import jax.numpy as jnp
from jax import lax
from jax.experimental import pallas as pl
from jax.experimental.pallas import tpu as pltpu

N_DEV = 32
M, NCOLS = 4096, 8192
HALF = NCOLS // 2
QTR = HALF // 2
CHUNK = M // N_DEV


def _logical_order():
    order = []
    for z in range(4):
        for yi in range(4):
            xs = (0, 1) if yi % 2 == 0 else (1, 0)
            for x in xs:
                order.append((x, yi, z))
    return order


def _ring_order():
    half = []
    for zi in range(4):
        ys = range(4) if zi % 2 == 0 else range(3, -1, -1)
        for y in ys:
            half.append((0, y, zi))
    return half + [(1, y, z) for (_, y, z) in reversed(half)]


_LOG = _logical_order()
_RING = _ring_order()
_L_OF_COORD = {c: l for l, c in enumerate(_LOG)}
_RING_L = [_L_OF_COORD[c] for c in _RING]
_RPOS = [0] * N_DEV
for p, l in enumerate(_RING_L):
    _RPOS[l] = p
_NEXT = [0] * N_DEV
_PREV = [0] * N_DEV
for p, l in enumerate(_RING_L):
    _NEXT[l] = _RING_L[(p + 1) % N_DEV]
    _PREV[l] = _RING_L[(p - 1) % N_DEV]


class _Dir:

    def __init__(self, r, to, fro, col_off, acc, recv, lc,
                 send_sems, recv_sems, lc_sem, out_sem, credit_sems):
        self.r = r
        self.to = to
        self.fro = fro
        self.col_off = col_off
        self.acc = acc
        self.recv = recv
        self.lc = lc
        self.send_sems = send_sems
        self.recv_sems = recv_sems
        self.lc_sem = lc_sem
        self.out_sem = out_sem
        self.credit_sems = credit_sems

    def rdma_sub(self, src, slot, sub):
        return pltpu.make_async_remote_copy(
            src_ref=src,
            dst_ref=self.recv.at[slot, :, pl.ds(sub * QTR, QTR)],
            send_sem=self.send_sems.at[slot, sub],
            recv_sem=self.recv_sems.at[slot, sub],
            device_id=(self.to,),
            device_id_type=pl.DeviceIdType.MESH,
        )

    def rdma_full(self, src, slot):
        return pltpu.make_async_remote_copy(
            src_ref=src,
            dst_ref=self.recv.at[slot],
            send_sem=self.send_sems.at[slot, 0],
            recv_sem=self.recv_sems.at[slot, 0],
            device_id=(self.to,),
            device_id_type=pl.DeviceIdType.MESH,
        )

    def credit(self, sub):
        pl.semaphore_signal(self.credit_sems.at[sub], inc=1,
                            device_id=(self.fro,),
                            device_id_type=pl.DeviceIdType.MESH)

    def credit_wait(self, sub):
        pl.semaphore_wait(self.credit_sems.at[sub], 1)


def _body(part_ref, r_ref, nxt_ref, prv_ref, out_ref, amax_ref,
          acc_f, recv_f, lc_f, acc_b, recv_b, lc_b,
          send_f, recv_sf, lc_sf, out_sf, cred_f,
          send_b, recv_sb, lc_sb, out_sb, cred_b):
    r = r_ref[0]
    nxt = nxt_ref[0]
    prv = prv_ref[0]
    r_b = (N_DEV - r) % N_DEV

    F = _Dir(r, nxt, prv, 0, acc_f, recv_f, lc_f,
             send_f, recv_sf, lc_sf, out_sf, cred_f)
    B = _Dir(r_b, prv, nxt, HALF, acc_b, recv_b, lc_b,
             send_b, recv_sb, lc_sb, out_sb, cred_b)
    dirs = (F, B)
    SUBS = (0, 1)

    def part_slice(d, c):
        return part_ref.at[pl.ds(c * CHUNK, CHUNK), pl.ds(d.col_off, HALF)]

    pres, lds = [], []
    for d in dirs:
        pre = pltpu.make_async_copy(part_slice(d, (d.r - 1) % N_DEV),
                                    d.acc.at[0], d.out_sem)
        pre.start()
        pres.append(pre)
        ld = pltpu.make_async_copy(part_slice(d, (d.r - 2) % N_DEV),
                                   d.lc.at[0], d.lc_sem)
        ld.start()
        lds.append(ld)

    barrier_sem = pltpu.get_barrier_semaphore()
    pl.semaphore_signal(barrier_sem, inc=1, device_id=(nxt,),
                        device_id_type=pl.DeviceIdType.MESH)
    pl.semaphore_signal(barrier_sem, inc=1, device_id=(prv,),
                        device_id_type=pl.DeviceIdType.MESH)
    pl.semaphore_wait(barrier_sem, 2)

    rd = {}
    for d, pre in zip(dirs, pres):
        pre.wait()
        for sub in SUBS:
            rm = d.rdma_sub(d.acc.at[0, :, pl.ds(sub * QTR, QTR)], 0, sub)
            rm.start()
            rd[(id(d), sub)] = rm
    ld_cur = lds

    for s in range(N_DEV - 1):
        slot = s % 2
        nslot = 1 - slot
        ld_next = []
        for di, d in enumerate(dirs):
            for sub in SUBS:
                cols = pl.ds(sub * QTR, QTR)
                rd[(id(d), sub)].wait()
                if sub == 0:
                    ld_cur[di].wait()
                if s < N_DEV - 2:
                    d.acc[nslot, :, sub * QTR:(sub + 1) * QTR] = (
                        d.recv[slot, :, sub * QTR:(sub + 1) * QTR]
                        + d.lc[slot, :, sub * QTR:(sub + 1) * QTR])
                else:
                    d.acc[nslot, :, sub * QTR:(sub + 1) * QTR] = jnp.maximum(
                        d.recv[slot, :, sub * QTR:(sub + 1) * QTR]
                        + d.lc[slot, :, sub * QTR:(sub + 1) * QTR], 0.0)
                d.credit(sub)
                if s < N_DEV - 2:
                    if s >= 1:
                        d.credit_wait(sub)
                    rm = d.rdma_sub(d.acc.at[nslot, :, cols], nslot, sub)
                    rm.start()
                    rd[(id(d), sub)] = rm
                if sub == 1 and s < N_DEV - 2:
                    ld = pltpu.make_async_copy(
                        part_slice(d, (d.r - 3 - s) % N_DEV),
                        d.lc.at[nslot], d.lc_sem)
                    ld.start()
                    ld_next.append(ld)
        ld_cur = ld_next

    amax_val = jnp.maximum(jnp.max(F.acc[1]), jnp.max(B.acc[1]))

    for t in range(N_DEV - 1):
        slot = (N_DEV - 1 + t) % 2
        for d in dirs:
            for sub in SUBS:
                d.credit_wait(sub)
        rdmas = []
        for d in dirs:
            src = d.acc.at[1] if t == 0 else d.recv.at[1 - slot]
            rm = d.rdma_full(src, slot)
            rm.start()
            rdmas.append(rm)
        sts = []
        for d in dirs:
            if t == 0:
                src, row = d.acc.at[1], d.r * CHUNK
            else:
                origin = (d.r - t) % N_DEV
                src, row = d.recv.at[1 - slot], origin * CHUNK
                amax_val = jnp.maximum(amax_val, jnp.max(d.recv[1 - slot]))
            st = pltpu.make_async_copy(
                src, out_ref.at[pl.ds(row, CHUNK), pl.ds(d.col_off, HALF)],
                d.out_sem)
            st.start()
            sts.append(st)
        for st in sts:
            st.wait()
        for d, rm in zip(dirs, rdmas):
            rm.wait()
            if t > 0:
                for sub in SUBS:
                    d.credit(sub)
    slot = (2 * N_DEV - 3) % 2
    sts = []
    for d in dirs:
        origin = (d.r - (N_DEV - 1)) % N_DEV
        st = pltpu.make_async_copy(
            d.recv.at[slot],
            out_ref.at[pl.ds(origin * CHUNK, CHUNK), pl.ds(d.col_off, HALF)],
            d.out_sem)
        st.start()
        amax_val = jnp.maximum(amax_val, jnp.max(d.recv[slot]))
        sts.append(st)
    for st, d in zip(sts, dirs):
        st.wait()
        for sub in SUBS:
            d.credit(sub)

    amax_ref[0, 0] = amax_val
    for d in dirs:
        for sub in SUBS:
            pl.semaphore_wait(d.credit_sems.at[sub], 2)


def _all_reduce_relu_amax(partial, r, nxt, prv):
    dir_scratch = [
        pltpu.VMEM((2, CHUNK, HALF), jnp.float32),
        pltpu.VMEM((2, CHUNK, HALF), jnp.float32),
        pltpu.VMEM((2, CHUNK, HALF), jnp.float32),
    ]
    dir_sems = [
        pltpu.SemaphoreType.DMA((2, 2)),
        pltpu.SemaphoreType.DMA((2, 2)),
        pltpu.SemaphoreType.DMA,
        pltpu.SemaphoreType.DMA,
        pltpu.SemaphoreType.REGULAR((2,)),
    ]
    return pl.pallas_call(
        _body,
        out_shape=[
            jax.ShapeDtypeStruct((M, NCOLS), jnp.float32),
            jax.ShapeDtypeStruct((1, 1), jnp.float32),
        ],
        in_specs=[
            pl.BlockSpec(memory_space=pl.ANY),
            pl.BlockSpec(memory_space=pltpu.SMEM),
            pl.BlockSpec(memory_space=pltpu.SMEM),
            pl.BlockSpec(memory_space=pltpu.SMEM),
        ],
        out_specs=[
            pl.BlockSpec(memory_space=pl.ANY),
            pl.BlockSpec(memory_space=pltpu.SMEM),
        ],
        scratch_shapes=(dir_scratch + dir_scratch
                        + dir_sems + dir_sems),
        compiler_params=pltpu.CompilerParams(collective_id=0),
    )(partial, r, nxt, prv)


def kernel(x, w_mat):
    partial = lax.dot_general(
        x, w_mat, (((1,), (0,)), ((), ())),
        precision=lax.Precision.HIGHEST,
        preferred_element_type=jnp.float32,
    )
    i = lax.axis_index("i")
    r = jnp.asarray(_RPOS, jnp.int32)[i].reshape(1)
    nxt = jnp.asarray(_NEXT, jnp.int32)[i].reshape(1)
    prv = jnp.asarray(_PREV, jnp.int32)[i].reshape(1)
    y, amax = _all_reduce_relu_amax(partial, r, nxt, prv)

    scale = amax[0, 0] / 448.0
    v = jnp.minimum(y / scale, 448.0)
    bits = lax.bitcast_convert_type(v, jnp.int32)
    rb = (bits + 0x7FFFF + ((bits >> 20) & 1)) & ~0xFFFFF
    norm = lax.bitcast_convert_type(rb, jnp.float32)
    sub = jnp.round(v * 512.0) * jnp.float32(1.0 / 512.0)
    q = jnp.where(v < 2.0 ** -6, sub, norm)
    return q * scale
